# baseline (speedup 1.0000x reference)
import functools
import jax
import jax.numpy as jnp
from jax import lax
from jax.experimental import pallas as pl
from jax.experimental.pallas import tpu as pltpu, tpu_sc as plsc

L, B, T, D = 4, 16, 1024, 768
NC, NS = 2, 16
TH = T // 2
CH = 64
NCHUNK = L * (TH // CH)
DJ = D // 16

_DN = lax.GatherDimensionNumbers(
    offset_dims=(), collapsed_slice_dims=(0,), start_index_map=(0,))


def _dgather(v, idx):
    return lax.gather(v, idx[:, None], _DN, slice_sizes=(1,),
                      mode=lax.GatherScatterMode.PROMISE_IN_BOUNDS)


def _sc_body(pt_hbm, am_hbm, out_hbm, buf0, buf1, ab0, ab1, ab2, ab3,
             wbuf, accbuf, sem0, sem1):
    c = lax.axis_index("c")
    s = lax.axis_index("s")
    wid = s * NC + c
    b = wid // 2
    h = wid % 2
    t0 = h * TH

    abufs = (ab0, ab1, ab2, ab3)
    for m in range(L):
        pltpu.sync_copy(am_hbm.at[m, b, pl.ds(2 * t0, 2 * TH)], abufs[m])

    j16 = lax.iota(jnp.int32, 16)
    lo = j16 < 8
    idx_odd = 2 * (j16 & 7) + 1
    idx_even = 2 * (j16 & 7)

    def wbody(i, carry):
        base = i * 32
        acc = jnp.zeros((16,), jnp.float32)
        for m in range(L):
            v0 = abufs[m][pl.ds(base, 16)]
            v1 = abufs[m][pl.ds(base + 16, 16)]
            odd = jnp.where(lo, _dgather(v0, idx_odd), _dgather(v1, idx_odd))
            even = jnp.where(lo, _dgather(v0, idx_even), _dgather(v1, idx_even))
            acc = acc + (odd - even)
        w = 1.0 / (1.0 + jnp.exp(acc * (-1.0 / L)))
        wbuf[pl.ds(i * 16, 16)] = w
        return carry

    lax.fori_loop(0, TH // 16, wbody, 0)

    bufs = (buf0, buf1)
    sems = (sem0, sem1)

    def src(i):
        l, k = divmod(i, TH // CH)
        return pt_hbm.at[l, b, pl.ds(t0 + k * CH, CH)]

    def process(buf, k, acc):
        def row_body(r, a):
            t_loc = k * CH + r
            wv0 = wbuf[pl.ds((t_loc // 16) * 16, 16)]
            wv = _dgather(wv0, jnp.full((16,), t_loc % 16, jnp.int32))
            return tuple(a[j] + wv * buf[r, pl.ds(16 * j, 16)]
                         for j in range(DJ))
        return lax.fori_loop(0, CH, row_body, acc)

    acc = tuple(jnp.zeros((16,), jnp.float32) for _ in range(DJ))
    copies = [None, None]
    copies[0] = pltpu.make_async_copy(src(0), bufs[0], sems[0])
    copies[0].start()
    for i in range(NCHUNK):
        if i + 1 < NCHUNK:
            copies[(i + 1) % 2] = pltpu.make_async_copy(
                src(i + 1), bufs[(i + 1) % 2], sems[(i + 1) % 2])
            copies[(i + 1) % 2].start()
        copies[i % 2].wait()
        acc = process(bufs[i % 2], i % (TH // CH), acc)

    for j in range(DJ):
        accbuf[pl.ds(16 * j, 16)] = acc[j]
    pltpu.sync_copy(accbuf, out_hbm.at[h, b])


def _sc_pool(patch_tokens, anomaly_maps):
    mesh = plsc.VectorSubcoreMesh(core_axis_name="c", subcore_axis_name="s",
                                  num_cores=NC, num_subcores=NS)
    f = pl.kernel(
        _sc_body,
        out_type=jax.ShapeDtypeStruct((2, B, D), jnp.float32),
        mesh=mesh,
        scratch_types=[
            pltpu.VMEM((CH, D), jnp.float32),
            pltpu.VMEM((CH, D), jnp.float32),
            pltpu.VMEM((2 * TH,), jnp.float32),
            pltpu.VMEM((2 * TH,), jnp.float32),
            pltpu.VMEM((2 * TH,), jnp.float32),
            pltpu.VMEM((2 * TH,), jnp.float32),
            pltpu.VMEM((TH,), jnp.float32),
            pltpu.VMEM((D,), jnp.float32),
            pltpu.SemaphoreType.DMA,
            pltpu.SemaphoreType.DMA,
        ],
    )
    return f(patch_tokens, anomaly_maps.reshape(L, B, 2 * T))


def _tc_finish_body(p_ref, out_ref):
    sm = (p_ref[0] + p_ref[1]) * (1.0 / L)
    n = jnp.sqrt(jnp.sum(sm * sm, axis=1, keepdims=True))
    out_ref[...] = sm / jnp.maximum(n, 1e-12)


def kernel(patch_tokens, anomaly_maps):
    partials = _sc_pool(patch_tokens, anomaly_maps)
    out = pl.pallas_call(
        _tc_finish_body,
        out_shape=jax.ShapeDtypeStruct((B, D), jnp.float32),
    )(partials)
    return out


# speedup vs baseline: 1.3223x; 1.3223x over previous
import functools
import jax
import jax.numpy as jnp
from jax import lax
from jax.experimental import pallas as pl
from jax.experimental.pallas import tpu as pltpu, tpu_sc as plsc

L, B, T, D = 4, 16, 1024, 768
L_SC = 3
NC, NS = 2, 16
TH = T // 2
CH = 64
NCHUNK = TH // CH
DJ = D // 16

_DN = lax.GatherDimensionNumbers(
    offset_dims=(), collapsed_slice_dims=(0,), start_index_map=(0,))


def _dgather(v, idx):
    return lax.gather(v, idx[:, None], _DN, slice_sizes=(1,),
                      mode=lax.GatherScatterMode.PROMISE_IN_BOUNDS)


def _sc_body(pt_hbm, am_hbm, out_hbm, buf0, buf1, ab0, ab1, ab2, ab3,
             wbuf, accbuf, sem0, sem1):
    c = lax.axis_index("c")
    s = lax.axis_index("s")
    wid = s * NC + c
    b = wid // 2
    h = wid % 2
    t0 = h * TH

    abufs = (ab0, ab1, ab2, ab3)
    for m in range(L):
        pltpu.sync_copy(am_hbm.at[m, b, pl.ds(2 * t0, 2 * TH)], abufs[m])

    j16 = lax.iota(jnp.int32, 16)
    lo = j16 < 8
    idx_odd = 2 * (j16 & 7) + 1
    idx_even = 2 * (j16 & 7)

    def wbody(i, carry):
        base = i * 32
        acc = jnp.zeros((16,), jnp.float32)
        for m in range(L):
            v0 = abufs[m][pl.ds(base, 16)]
            v1 = abufs[m][pl.ds(base + 16, 16)]
            odd = jnp.where(lo, _dgather(v0, idx_odd), _dgather(v1, idx_odd))
            even = jnp.where(lo, _dgather(v0, idx_even), _dgather(v1, idx_even))
            acc = acc + (odd - even)
        w = 1.0 / (1.0 + jnp.exp(acc * (-1.0 / L)))
        wbuf[pl.ds(i * 16, 16)] = w
        return carry

    lax.fori_loop(0, TH // 16, wbody, 0)

    bufs = (buf0, buf1)
    sems = (sem0, sem1)

    def src(i):
        return pt_hbm.at[L_SC, b, pl.ds(t0 + i * CH, CH)]

    def process(buf, k, acc):
        def row_body(r, a):
            t_loc = k * CH + r
            wv0 = wbuf[pl.ds((t_loc // 16) * 16, 16)]
            wv = _dgather(wv0, jnp.full((16,), t_loc % 16, jnp.int32))
            return tuple(a[j] + wv * buf[r, pl.ds(16 * j, 16)]
                         for j in range(DJ))
        return lax.fori_loop(0, CH, row_body, acc)

    acc = tuple(jnp.zeros((16,), jnp.float32) for _ in range(DJ))
    copies = [None, None]
    copies[0] = pltpu.make_async_copy(src(0), bufs[0], sems[0])
    copies[0].start()
    for i in range(NCHUNK):
        if i + 1 < NCHUNK:
            copies[(i + 1) % 2] = pltpu.make_async_copy(
                src(i + 1), bufs[(i + 1) % 2], sems[(i + 1) % 2])
            copies[(i + 1) % 2].start()
        copies[i % 2].wait()
        acc = process(bufs[i % 2], i, acc)

    for j in range(DJ):
        accbuf[pl.ds(16 * j, 16)] = acc[j]
    pltpu.sync_copy(accbuf, out_hbm.at[h, b])


def _sc_pool(patch_tokens, am_flat):
    mesh = plsc.VectorSubcoreMesh(core_axis_name="c", subcore_axis_name="s",
                                  num_cores=NC, num_subcores=NS)
    f = pl.kernel(
        _sc_body,
        out_type=jax.ShapeDtypeStruct((2, B, D), jnp.float32),
        mesh=mesh,
        scratch_types=[
            pltpu.VMEM((CH, D), jnp.float32),
            pltpu.VMEM((CH, D), jnp.float32),
            pltpu.VMEM((2 * TH,), jnp.float32),
            pltpu.VMEM((2 * TH,), jnp.float32),
            pltpu.VMEM((2 * TH,), jnp.float32),
            pltpu.VMEM((2 * TH,), jnp.float32),
            pltpu.VMEM((TH,), jnp.float32),
            pltpu.VMEM((D,), jnp.float32),
            pltpu.SemaphoreType.DMA,
            pltpu.SemaphoreType.DMA,
        ],
    )
    return f(patch_tokens, am_flat)


def _tc_body(am_ref, pt_ref, out_ref):
    a = am_ref[:, 0]
    d = a[:, 1, :] - a[:, 0, :]
    d = jnp.sum(d, axis=0, keepdims=True) * (1.0 / L)
    w = jax.nn.sigmoid(d)

    acc = jnp.zeros((1, pt_ref.shape[3]), dtype=jnp.float32)
    for l in range(L_SC):
        acc = acc + jnp.dot(w, pt_ref[l, 0], preferred_element_type=jnp.float32)
    out_ref[...] = acc[:, None, :]


def _tc_finish_body(tc_ref, sc_ref, out_ref):
    sm = (tc_ref[:, 0, :] + sc_ref[0] + sc_ref[1]) * (1.0 / L)
    n = jnp.sqrt(jnp.sum(sm * sm, axis=1, keepdims=True))
    out_ref[...] = sm / jnp.maximum(n, 1e-12)


def kernel(patch_tokens, anomaly_maps):
    am_t = jnp.swapaxes(anomaly_maps, 2, 3)
    am_flat = anomaly_maps.reshape(L, B, 2 * T)

    sc_part = _sc_pool(patch_tokens, am_flat)

    tc_part = pl.pallas_call(
        _tc_body,
        grid=(B,),
        in_specs=[
            pl.BlockSpec((L, 1, 2, T), lambda b: (0, b, 0, 0)),
            pl.BlockSpec((L_SC, 1, T, D), lambda b: (0, b, 0, 0)),
        ],
        out_specs=pl.BlockSpec((1, 1, D), lambda b: (b, 0, 0)),
        out_shape=jax.ShapeDtypeStruct((B, 1, D), jnp.float32),
    )(am_t, patch_tokens)

    out = pl.pallas_call(
        _tc_finish_body,
        out_shape=jax.ShapeDtypeStruct((B, D), jnp.float32),
    )(tc_part, sc_part)
    return out
